# baseline (speedup 1.0000x reference)
import functools

import jax
import jax.numpy as jnp
from jax import lax
from jax.experimental import pallas as pl
from jax.experimental.pallas import tpu as pltpu
from jax.experimental.pallas import tpu_sc as plsc

N = 10000
E = 320000
D = 128
NC = 2
NS = 16
EPT = E // NS
C = 80
NSLOT = 2
NGROUP = EPT // (C * NSLOT)
OWN = 624
NRB = OWN // C
RBL = OWN - NRB * C
REM = N - NS * OWN
CW = 16

_f32 = jnp.float32
_i32 = jnp.int32

_PIPE = True


def _sc_agg_body(with_gather, tabs, srci, dsti, zrow,
                 sums, acc, rows, sidx, didx, isem, gsem, ssem):
    cid = lax.axis_index("c")
    sid = lax.axis_index("s")
    row0 = sid * OWN
    obase = cid * N
    ebase = cid * E + sid * EPT

    pltpu.sync_copy(zrow, rows[0])
    if not with_gather:
        pltpu.sync_copy(srci, rows[1])
    for k in range(NRB):
        pltpu.sync_copy(rows[0], acc.at[pl.ds(row0 + k * C, C), :])
    rlo = row0 + NRB * C
    pltpu.sync_copy(rows[0].at[pl.ds(0, RBL), :], acc.at[pl.ds(rlo, RBL), :])

    @pl.when(sid == 0)
    def _():
        pltpu.sync_copy(rows[0].at[pl.ds(0, REM), :],
                        acc.at[pl.ds(NS * OWN, REM), :])

    plsc.subcore_barrier()

    def issue_idx(q, jj):
        off = ebase + jj * C
        if with_gather:
            pltpu.async_copy(srci.at[pl.ds(off, C)], sidx[q], isem[q])
        pltpu.async_copy(dsti.at[pl.ds(off, C)], didx[q], isem[q])

    def wait_idx(q):
        if with_gather:
            pltpu.make_async_copy(srci.at[pl.ds(0, C)], sidx[q],
                                  isem[q]).wait()
        pltpu.make_async_copy(dsti.at[pl.ds(0, C)], didx[q], isem[q]).wait()

    def issue_gather(s, q):
        if with_gather:
            pltpu.async_copy(tabs.at[sidx[q]], rows[s], gsem[s])

    def wait_gather(s, q):
        if with_gather:
            pltpu.make_async_copy(tabs.at[sidx[q]], rows[s], gsem[s]).wait()

    def srcbuf(s):
        return rows[s] if with_gather else rows[1]

    def issue_scatter(s, q):
        pltpu.async_copy(srcbuf(s), acc.at[didx[q]], ssem[s], add=True)

    def wait_scatter(s, q):
        pltpu.make_async_copy(srcbuf(s), acc.at[didx[q]], ssem[s]).wait()

    if not _PIPE:
        def chunk_body(j, carry):
            issue_idx(0, j)
            wait_idx(0)
            issue_gather(0, 0)
            wait_gather(0, 0)
            issue_scatter(0, 0)
            wait_scatter(0, 0)
            return carry

        lax.fori_loop(0, EPT // C, chunk_body, 0)
    else:
        for s in range(NSLOT):
            issue_idx(s, s)

        def one_group(g, parity):
            qs = [parity * NSLOT + s for s in range(NSLOT)]
            qo = [(1 - parity) * NSLOT + s for s in range(NSLOT)]
            for s in range(NSLOT):
                @pl.when(g > 0)
                def _(s=s):
                    wait_scatter(s, qo[s])
                wait_idx(qs[s])
                issue_gather(s, qs[s])
            for s in range(NSLOT):
                wait_gather(s, qs[s])
                issue_scatter(s, qs[s])
                @pl.when(g + 1 < NGROUP)
                def _(s=s, g=g):
                    issue_idx(qo[s], (g + 1) * NSLOT + s)

        def loop_body(t, carry):
            one_group(2 * t, 0)
            one_group(2 * t + 1, 1)
            return carry

        lax.fori_loop(0, NGROUP // 2, loop_body, 0)
        if NGROUP % 2:
            one_group(NGROUP - 1, 0)
        lastp = (NGROUP - 1) % 2
        for s in range(NSLOT):
            wait_scatter(s, lastp * NSLOT + s)

    plsc.subcore_barrier()

    for k in range(NRB):
        r = row0 + k * C
        pltpu.sync_copy(acc.at[pl.ds(r, C), :], rows[0])
        pltpu.sync_copy(rows[0], sums.at[pl.ds(obase + r, C), :])
    pltpu.sync_copy(acc.at[pl.ds(rlo, RBL), :], rows[0].at[pl.ds(0, RBL), :])
    pltpu.sync_copy(rows[0].at[pl.ds(0, RBL), :],
                    sums.at[pl.ds(obase + rlo, RBL), :])

    @pl.when(sid == 0)
    def _():
        r = NS * OWN
        pltpu.sync_copy(acc.at[pl.ds(r, REM), :], rows[0].at[pl.ds(0, REM), :])
        pltpu.sync_copy(rows[0].at[pl.ds(0, REM), :],
                        sums.at[pl.ds(obase + r, REM), :])


def _make_sc_kernel(with_gather):
    out_type = jax.ShapeDtypeStruct((NC * N, D), _f32)
    scratch = [pltpu.VMEM_SHARED((N, D), _f32)]
    scratch += [pltpu.VMEM((C, D), _f32) for _ in range(NSLOT)]
    scratch += [pltpu.VMEM((C,), _i32) for _ in range(2 * NSLOT)]
    scratch += [pltpu.VMEM((C,), _i32) for _ in range(2 * NSLOT)]
    scratch += [pltpu.SemaphoreType.DMA for _ in range(2 * NSLOT)]
    scratch += [pltpu.SemaphoreType.DMA for _ in range(NSLOT)]
    scratch += [pltpu.SemaphoreType.DMA for _ in range(NSLOT)]

    def body(*refs):
        tabs, srci, dsti, zrow, sums = refs[:5]
        k = 5
        acc = refs[k]; k += 1
        rows = list(refs[k:k + NSLOT]); k += NSLOT
        sidx = list(refs[k:k + 2 * NSLOT]); k += 2 * NSLOT
        didx = list(refs[k:k + 2 * NSLOT]); k += 2 * NSLOT
        isem = list(refs[k:k + 2 * NSLOT]); k += 2 * NSLOT
        gsem = list(refs[k:k + NSLOT]); k += NSLOT
        ssem = list(refs[k:k + NSLOT]); k += NSLOT
        _sc_agg_body(with_gather, tabs, srci, dsti, zrow,
                     sums, acc, rows, sidx, didx, isem, gsem, ssem)

    mesh = plsc.VectorSubcoreMesh(core_axis_name="c", subcore_axis_name="s",
                                  num_cores=NC, num_subcores=NS)
    return pl.kernel(body, out_type=out_type, mesh=mesh, scratch_types=scratch)


_sc_agg = _make_sc_kernel(True)
_sc_count = _make_sc_kernel(False)


def _tc_body(x_ref, s_ref, c_ref, ws_ref, bs_ref, wn_ref, bn_ref, o_ref, *, relu):
    cnt = c_ref[:, 0:1]
    inv = 1.0 / jnp.maximum(cnt, 1.0)
    mean = s_ref[...] * inv
    acc = (jnp.dot(x_ref[...], ws_ref[...], preferred_element_type=_f32)
           + jnp.dot(mean, wn_ref[...], preferred_element_type=_f32)
           + bs_ref[...] + bn_ref[...])
    if relu:
        acc = jnp.maximum(acc, 0.0)
    o_ref[...] = acc


def _tc_layer(x, s, cnt, ws, bs, wn, bn, relu):
    R = 1000
    W = ws.shape[1]
    grid = (N // R,)
    return pl.pallas_call(
        functools.partial(_tc_body, relu=relu),
        grid=grid,
        in_specs=[
            pl.BlockSpec((R, D), lambda i: (i, 0)),
            pl.BlockSpec((R, D), lambda i: (i, 0)),
            pl.BlockSpec((R, CW), lambda i: (i, 0)),
            pl.BlockSpec((D, W), lambda i: (0, 0)),
            pl.BlockSpec((1, W), lambda i: (0, 0)),
            pl.BlockSpec((D, W), lambda i: (0, 0)),
            pl.BlockSpec((1, W), lambda i: (0, 0)),
        ],
        out_specs=pl.BlockSpec((R, W), lambda i: (i, 0)),
        out_shape=jax.ShapeDtypeStruct((N, W), _f32),
    )(x, s, cnt, ws, bs.reshape(1, W), wn, bn.reshape(1, W))


def kernel(x_user, x_item, edge_index_clicks, edge_index_clicked_by,
           c1ck_Ws, c1ck_bs, c1ck_Wn, c1ck_bn,
           c1cb_Ws, c1cb_bs, c1cb_Wn, c1cb_bn,
           c2ck_Ws, c2ck_bs, c2ck_Wn, c2ck_bn,
           c2cb_Ws, c2cb_bs, c2cb_Wn, c2cb_bn):
    srci = jnp.concatenate(
        [edge_index_clicks[0], edge_index_clicked_by[0] + N]).astype(_i32)
    dsti = jnp.concatenate(
        [edge_index_clicks[1], edge_index_clicked_by[1]]).astype(_i32)
    zrow = jnp.zeros((C, D), _f32)
    orow = jnp.ones((C, D), _f32)

    csum = _sc_count(zrow, orow, dsti, zrow)
    cnt_item = csum[:N, :CW]
    cnt_user = csum[N:, :CW]

    tabs1 = jnp.concatenate([x_user, x_item], axis=0)
    sums1 = _sc_agg(tabs1, srci, dsti, zrow)

    h_item = _tc_layer(x_item, sums1[:N], cnt_item,
                       c1ck_Ws, c1ck_bs, c1ck_Wn, c1ck_bn, relu=True)
    h_user = _tc_layer(x_user, sums1[N:], cnt_user,
                       c1cb_Ws, c1cb_bs, c1cb_Wn, c1cb_bn, relu=True)

    tabs2 = jnp.concatenate([h_user, h_item], axis=0)
    sums2 = _sc_agg(tabs2, srci, dsti, zrow)

    out_item = _tc_layer(h_item, sums2[:N], cnt_item,
                         c2ck_Ws, c2ck_bs, c2ck_Wn, c2ck_bn, relu=False)
    out_user = _tc_layer(h_user, sums2[N:], cnt_user,
                         c2cb_Ws, c2cb_bs, c2cb_Wn, c2cb_bn, relu=False)
    return (out_user, out_item)


# speedup vs baseline: 1.1951x; 1.1951x over previous
import functools

import jax
import jax.numpy as jnp
from jax import lax
from jax.experimental import pallas as pl
from jax.experimental.pallas import tpu as pltpu
from jax.experimental.pallas import tpu_sc as plsc

N = 10000
E = 320000
D = 128
NC = 2
NS = 16
EPT = E // NS
C = 40
NSLOT = 5
NGROUP = EPT // (C * NSLOT)
OWN = 624
NRB = OWN // C
RBL = OWN - NRB * C
REM = N - NS * OWN
CW = 16

_f32 = jnp.float32
_i32 = jnp.int32

def _sc_agg_body(with_gather, W, tabs, srci, dsti, zrow,
                 sums, acc, rows, sidx, didx, isem, gsem, ssem):
    cid = lax.axis_index("c")
    sid = lax.axis_index("s")
    row0 = sid * OWN
    obase = cid * N
    ebase = cid * E + sid * EPT

    pltpu.sync_copy(zrow, rows[0])
    if not with_gather:
        pltpu.sync_copy(srci, rows[1])
    for k in range(NRB):
        pltpu.sync_copy(rows[0], acc.at[pl.ds(row0 + k * C, C), :])
    rlo = row0 + NRB * C
    pltpu.sync_copy(rows[0].at[pl.ds(0, RBL), :], acc.at[pl.ds(rlo, RBL), :])

    @pl.when(sid == 0)
    def _():
        pltpu.sync_copy(rows[0].at[pl.ds(0, REM), :],
                        acc.at[pl.ds(NS * OWN, REM), :])

    plsc.subcore_barrier()

    def issue_idx(q, jj):
        off = ebase + jj * C
        if with_gather:
            pltpu.async_copy(srci.at[pl.ds(off, C)], sidx[q], isem[q])
        pltpu.async_copy(dsti.at[pl.ds(off, C)], didx[q], isem[q])

    def wait_idx(q):
        if with_gather:
            pltpu.make_async_copy(srci.at[pl.ds(0, C)], sidx[q],
                                  isem[q]).wait()
        pltpu.make_async_copy(dsti.at[pl.ds(0, C)], didx[q], isem[q]).wait()

    def issue_gather(s, q):
        if with_gather:
            pltpu.async_copy(tabs.at[sidx[q]], rows[s], gsem[s])

    def wait_gather(s, q):
        if with_gather:
            pltpu.make_async_copy(tabs.at[sidx[q]], rows[s], gsem[s]).wait()

    def srcbuf(s):
        return rows[s] if with_gather else rows[1]

    def issue_scatter(s, q):
        pltpu.async_copy(srcbuf(s), acc.at[didx[q]], ssem[s], add=True)

    def wait_scatter(s, q):
        pltpu.make_async_copy(srcbuf(s), acc.at[didx[q]], ssem[s]).wait()

    if True:
        for s in range(NSLOT):
            issue_idx(s, s)

        def one_group(g, parity):
            qs = [parity * NSLOT + s for s in range(NSLOT)]
            qo = [(1 - parity) * NSLOT + s for s in range(NSLOT)]
            for s in range(NSLOT):
                @pl.when(g > 0)
                def _(s=s):
                    wait_scatter(s, qo[s])
                wait_idx(qs[s])
                issue_gather(s, qs[s])
            for s in range(NSLOT):
                wait_gather(s, qs[s])
                issue_scatter(s, qs[s])
                @pl.when(g + 1 < NGROUP)
                def _(s=s, g=g):
                    issue_idx(qo[s], (g + 1) * NSLOT + s)

        def loop_body(t, carry):
            one_group(2 * t, 0)
            one_group(2 * t + 1, 1)
            return carry

        lax.fori_loop(0, NGROUP // 2, loop_body, 0)
        if NGROUP % 2:
            one_group(NGROUP - 1, 0)
        lastp = (NGROUP - 1) % 2
        for s in range(NSLOT):
            wait_scatter(s, lastp * NSLOT + s)

    plsc.subcore_barrier()

    for k in range(NRB):
        r = row0 + k * C
        pltpu.sync_copy(acc.at[pl.ds(r, C), :], rows[0])
        pltpu.sync_copy(rows[0], sums.at[pl.ds(obase + r, C), :])
    pltpu.sync_copy(acc.at[pl.ds(rlo, RBL), :], rows[0].at[pl.ds(0, RBL), :])
    pltpu.sync_copy(rows[0].at[pl.ds(0, RBL), :],
                    sums.at[pl.ds(obase + rlo, RBL), :])

    @pl.when(sid == 0)
    def _():
        r = NS * OWN
        pltpu.sync_copy(acc.at[pl.ds(r, REM), :], rows[0].at[pl.ds(0, REM), :])
        pltpu.sync_copy(rows[0].at[pl.ds(0, REM), :],
                        sums.at[pl.ds(obase + r, REM), :])


def _make_sc_kernel(with_gather, W):
    out_type = jax.ShapeDtypeStruct((NC * N, W), _f32)
    scratch = [pltpu.VMEM_SHARED((N, W), _f32)]
    scratch += [pltpu.VMEM((C, W), _f32) for _ in range(NSLOT)]
    scratch += [pltpu.VMEM((C,), _i32) for _ in range(2 * NSLOT)]
    scratch += [pltpu.VMEM((C,), _i32) for _ in range(2 * NSLOT)]
    scratch += [pltpu.SemaphoreType.DMA for _ in range(2 * NSLOT)]
    scratch += [pltpu.SemaphoreType.DMA for _ in range(NSLOT)]
    scratch += [pltpu.SemaphoreType.DMA for _ in range(NSLOT)]

    def body(*refs):
        tabs, srci, dsti, zrow, sums = refs[:5]
        k = 5
        acc = refs[k]; k += 1
        rows = list(refs[k:k + NSLOT]); k += NSLOT
        sidx = list(refs[k:k + 2 * NSLOT]); k += 2 * NSLOT
        didx = list(refs[k:k + 2 * NSLOT]); k += 2 * NSLOT
        isem = list(refs[k:k + 2 * NSLOT]); k += 2 * NSLOT
        gsem = list(refs[k:k + NSLOT]); k += NSLOT
        ssem = list(refs[k:k + NSLOT]); k += NSLOT
        _sc_agg_body(with_gather, W, tabs, srci, dsti, zrow,
                     sums, acc, rows, sidx, didx, isem, gsem, ssem)

    mesh = plsc.VectorSubcoreMesh(core_axis_name="c", subcore_axis_name="s",
                                  num_cores=NC, num_subcores=NS)
    return pl.kernel(body, out_type=out_type, mesh=mesh, scratch_types=scratch)


_sc_agg = _make_sc_kernel(True, D)
_sc_count = _make_sc_kernel(False, D)


def _tc_body(x_ref, s_ref, c_ref, ws_ref, bs_ref, wn_ref, bn_ref, o_ref, *, relu):
    cnt = c_ref[:, 0:1]
    inv = 1.0 / jnp.maximum(cnt, 1.0)
    mean = s_ref[...] * inv
    acc = (jnp.dot(x_ref[...], ws_ref[...], preferred_element_type=_f32)
           + jnp.dot(mean, wn_ref[...], preferred_element_type=_f32)
           + bs_ref[...] + bn_ref[...])
    if relu:
        acc = jnp.maximum(acc, 0.0)
    o_ref[...] = acc


def _tc_layer(x, s, cnt, ws, bs, wn, bn, relu):
    R = 1000
    W = ws.shape[1]
    grid = (N // R,)
    return pl.pallas_call(
        functools.partial(_tc_body, relu=relu),
        grid=grid,
        in_specs=[
            pl.BlockSpec((R, D), lambda i: (i, 0)),
            pl.BlockSpec((R, D), lambda i: (i, 0)),
            pl.BlockSpec((R, CW), lambda i: (i, 0)),
            pl.BlockSpec((D, W), lambda i: (0, 0)),
            pl.BlockSpec((1, W), lambda i: (0, 0)),
            pl.BlockSpec((D, W), lambda i: (0, 0)),
            pl.BlockSpec((1, W), lambda i: (0, 0)),
        ],
        out_specs=pl.BlockSpec((R, W), lambda i: (i, 0)),
        out_shape=jax.ShapeDtypeStruct((N, W), _f32),
    )(x, s, cnt, ws, bs.reshape(1, W), wn, bn.reshape(1, W))


def kernel(x_user, x_item, edge_index_clicks, edge_index_clicked_by,
           c1ck_Ws, c1ck_bs, c1ck_Wn, c1ck_bn,
           c1cb_Ws, c1cb_bs, c1cb_Wn, c1cb_bn,
           c2ck_Ws, c2ck_bs, c2ck_Wn, c2ck_bn,
           c2cb_Ws, c2cb_bs, c2cb_Wn, c2cb_bn):
    srci = jnp.concatenate(
        [edge_index_clicks[0], edge_index_clicked_by[0] + N]).astype(_i32)
    dsti = jnp.concatenate(
        [edge_index_clicks[1], edge_index_clicked_by[1]]).astype(_i32)
    zrow = jnp.zeros((C, D), _f32)
    orow = jnp.ones((C, D), _f32)

    csum = _sc_count(zrow, orow, dsti, zrow)
    cnt_item = csum[:N, :CW]
    cnt_user = csum[N:, :CW]

    tabs1 = jnp.concatenate([x_user, x_item], axis=0)
    sums1 = _sc_agg(tabs1, srci, dsti, zrow)

    h_item = _tc_layer(x_item, sums1[:N], cnt_item,
                       c1ck_Ws, c1ck_bs, c1ck_Wn, c1ck_bn, relu=True)
    h_user = _tc_layer(x_user, sums1[N:], cnt_user,
                       c1cb_Ws, c1cb_bs, c1cb_Wn, c1cb_bn, relu=True)

    tabs2 = jnp.concatenate([h_user, h_item], axis=0)
    sums2 = _sc_agg(tabs2, srci, dsti, zrow)

    out_item = _tc_layer(h_item, sums2[:N], cnt_item,
                         c2ck_Ws, c2ck_bs, c2ck_Wn, c2ck_bn, relu=False)
    out_user = _tc_layer(h_user, sums2[N:], cnt_user,
                         c2cb_Ws, c2cb_bs, c2cb_Wn, c2cb_bn, relu=False)
    return (out_user, out_item)


# speedup vs baseline: 1.1958x; 1.0006x over previous
import functools

import jax
import jax.numpy as jnp
from jax import lax
from jax.experimental import pallas as pl
from jax.experimental.pallas import tpu as pltpu
from jax.experimental.pallas import tpu_sc as plsc

N = 10000
E = 320000
D = 128
NC = 2
NS = 16
EPT = E // NS
C = 40
NSLOT = 5
NGROUP = EPT // (C * NSLOT)
OWN = 624
NRB = OWN // C
RBL = OWN - NRB * C
REM = N - NS * OWN
CW = 16

_f32 = jnp.float32
_i32 = jnp.int32

def _sc_agg_body(with_gather, W, tabs, srci, dsti, zrow,
                 sums, acc, rows, sidx, didx, isem, gsem, ssem):
    cid = lax.axis_index("c")
    sid = lax.axis_index("s")
    row0 = sid * OWN
    obase = cid * N
    ebase = cid * E + sid * EPT

    pltpu.sync_copy(zrow, rows[0])
    if not with_gather:
        pltpu.sync_copy(srci, rows[1])
    for k in range(NRB):
        pltpu.sync_copy(rows[0], acc.at[pl.ds(row0 + k * C, C), :])
    rlo = row0 + NRB * C
    pltpu.sync_copy(rows[0].at[pl.ds(0, RBL), :], acc.at[pl.ds(rlo, RBL), :])

    @pl.when(sid == 0)
    def _():
        pltpu.sync_copy(rows[0].at[pl.ds(0, REM), :],
                        acc.at[pl.ds(NS * OWN, REM), :])

    plsc.subcore_barrier()

    def issue_idx(q, jj):
        off = ebase + jj * C
        if with_gather:
            pltpu.async_copy(srci.at[pl.ds(off, C)], sidx[q], isem[q])
        pltpu.async_copy(dsti.at[pl.ds(off, C)], didx[q], isem[q])

    def wait_idx(q):
        if with_gather:
            pltpu.make_async_copy(srci.at[pl.ds(0, C)], sidx[q],
                                  isem[q]).wait()
        pltpu.make_async_copy(dsti.at[pl.ds(0, C)], didx[q], isem[q]).wait()

    def issue_gather(s, q):
        if with_gather:
            pltpu.async_copy(tabs.at[sidx[q]], rows[s], gsem[s])

    def wait_gather(s, q):
        if with_gather:
            pltpu.make_async_copy(tabs.at[sidx[q]], rows[s], gsem[s]).wait()

    def srcbuf(s):
        return rows[s] if with_gather else rows[1]

    def issue_scatter(s, q):
        pltpu.async_copy(srcbuf(s), acc.at[didx[q]], ssem[s], add=True)

    def wait_scatter(s, q):
        pltpu.make_async_copy(srcbuf(s), acc.at[didx[q]], ssem[s]).wait()

    for s in range(NSLOT):
        issue_idx(s, s)

    def one_group(g, parity):
        qs = [parity * NSLOT + s for s in range(NSLOT)]
        qo = [(1 - parity) * NSLOT + s for s in range(NSLOT)]
        for s in range(NSLOT):
            @pl.when(g > 0)
            def _(s=s):
                wait_scatter(s, qo[s])
            wait_idx(qs[s])
            issue_gather(s, qs[s])
        for s in range(NSLOT):
            wait_gather(s, qs[s])
            issue_scatter(s, qs[s])
            @pl.when(g + 1 < NGROUP)
            def _(s=s, g=g):
                issue_idx(qo[s], (g + 1) * NSLOT + s)

    def loop_body(t, carry):
        one_group(2 * t, 0)
        one_group(2 * t + 1, 1)
        return carry

    lax.fori_loop(0, NGROUP // 2, loop_body, 0)
    if NGROUP % 2:
        one_group(NGROUP - 1, 0)
    lastp = (NGROUP - 1) % 2
    for s in range(NSLOT):
        wait_scatter(s, lastp * NSLOT + s)

    plsc.subcore_barrier()

    for k in range(NRB):
        r = row0 + k * C
        pltpu.sync_copy(acc.at[pl.ds(r, C), :], rows[0])
        pltpu.sync_copy(rows[0], sums.at[pl.ds(obase + r, C), :])
    pltpu.sync_copy(acc.at[pl.ds(rlo, RBL), :], rows[0].at[pl.ds(0, RBL), :])
    pltpu.sync_copy(rows[0].at[pl.ds(0, RBL), :],
                    sums.at[pl.ds(obase + rlo, RBL), :])

    @pl.when(sid == 0)
    def _():
        r = NS * OWN
        pltpu.sync_copy(acc.at[pl.ds(r, REM), :], rows[0].at[pl.ds(0, REM), :])
        pltpu.sync_copy(rows[0].at[pl.ds(0, REM), :],
                        sums.at[pl.ds(obase + r, REM), :])


def _make_sc_kernel(with_gather, W):
    out_type = jax.ShapeDtypeStruct((NC * N, W), _f32)
    scratch = [pltpu.VMEM_SHARED((N, W), _f32)]
    scratch += [pltpu.VMEM((C, W), _f32) for _ in range(NSLOT)]
    scratch += [pltpu.VMEM((C,), _i32) for _ in range(2 * NSLOT)]
    scratch += [pltpu.VMEM((C,), _i32) for _ in range(2 * NSLOT)]
    scratch += [pltpu.SemaphoreType.DMA for _ in range(2 * NSLOT)]
    scratch += [pltpu.SemaphoreType.DMA for _ in range(NSLOT)]
    scratch += [pltpu.SemaphoreType.DMA for _ in range(NSLOT)]

    def body(*refs):
        tabs, srci, dsti, zrow, sums = refs[:5]
        k = 5
        acc = refs[k]; k += 1
        rows = list(refs[k:k + NSLOT]); k += NSLOT
        sidx = list(refs[k:k + 2 * NSLOT]); k += 2 * NSLOT
        didx = list(refs[k:k + 2 * NSLOT]); k += 2 * NSLOT
        isem = list(refs[k:k + 2 * NSLOT]); k += 2 * NSLOT
        gsem = list(refs[k:k + NSLOT]); k += NSLOT
        ssem = list(refs[k:k + NSLOT]); k += NSLOT
        _sc_agg_body(with_gather, W, tabs, srci, dsti, zrow,
                     sums, acc, rows, sidx, didx, isem, gsem, ssem)

    mesh = plsc.VectorSubcoreMesh(core_axis_name="c", subcore_axis_name="s",
                                  num_cores=NC, num_subcores=NS)
    return pl.kernel(body, out_type=out_type, mesh=mesh, scratch_types=scratch)


_sc_agg = _make_sc_kernel(True, D)
_sc_count = _make_sc_kernel(False, D)


def _tc_body(x_ref, s_ref, c_ref, ws_ref, bs_ref, wn_ref, bn_ref, o_ref, *, relu):
    cnt = c_ref[:, 0:1]
    inv = 1.0 / jnp.maximum(cnt, 1.0)
    mean = s_ref[...] * inv
    acc = (jnp.dot(x_ref[...], ws_ref[...], preferred_element_type=_f32)
           + jnp.dot(mean, wn_ref[...], preferred_element_type=_f32)
           + bs_ref[...] + bn_ref[...])
    if relu:
        acc = jnp.maximum(acc, 0.0)
    o_ref[...] = acc


def _tc_layer(x, s, cnt, ws, bs, wn, bn, relu):
    R = 1000
    W = ws.shape[1]
    grid = (N // R,)
    return pl.pallas_call(
        functools.partial(_tc_body, relu=relu),
        grid=grid,
        in_specs=[
            pl.BlockSpec((R, D), lambda i: (i, 0)),
            pl.BlockSpec((R, D), lambda i: (i, 0)),
            pl.BlockSpec((R, CW), lambda i: (i, 0)),
            pl.BlockSpec((D, W), lambda i: (0, 0)),
            pl.BlockSpec((1, W), lambda i: (0, 0)),
            pl.BlockSpec((D, W), lambda i: (0, 0)),
            pl.BlockSpec((1, W), lambda i: (0, 0)),
        ],
        out_specs=pl.BlockSpec((R, W), lambda i: (i, 0)),
        out_shape=jax.ShapeDtypeStruct((N, W), _f32),
    )(x, s, cnt, ws, bs.reshape(1, W), wn, bn.reshape(1, W))


def kernel(x_user, x_item, edge_index_clicks, edge_index_clicked_by,
           c1ck_Ws, c1ck_bs, c1ck_Wn, c1ck_bn,
           c1cb_Ws, c1cb_bs, c1cb_Wn, c1cb_bn,
           c2ck_Ws, c2ck_bs, c2ck_Wn, c2ck_bn,
           c2cb_Ws, c2cb_bs, c2cb_Wn, c2cb_bn):
    srci = jnp.concatenate(
        [edge_index_clicks[0], edge_index_clicked_by[0] + N]).astype(_i32)
    dsti = jnp.concatenate(
        [edge_index_clicks[1], edge_index_clicked_by[1]]).astype(_i32)
    zrow = jnp.zeros((C, D), _f32)
    orow = jnp.ones((C, D), _f32)

    csum = _sc_count(zrow, orow, dsti, zrow)
    cnt_item = csum[:N, :CW]
    cnt_user = csum[N:, :CW]

    tabs1 = jnp.concatenate([x_user, x_item], axis=0)
    sums1 = _sc_agg(tabs1, srci, dsti, zrow)

    h_item = _tc_layer(x_item, sums1[:N], cnt_item,
                       c1ck_Ws, c1ck_bs, c1ck_Wn, c1ck_bn, relu=True)
    h_user = _tc_layer(x_user, sums1[N:], cnt_user,
                       c1cb_Ws, c1cb_bs, c1cb_Wn, c1cb_bn, relu=True)

    tabs2 = jnp.concatenate([h_user, h_item], axis=0)
    sums2 = _sc_agg(tabs2, srci, dsti, zrow)

    out_item = _tc_layer(h_item, sums2[:N], cnt_item,
                         c2ck_Ws, c2ck_bs, c2ck_Wn, c2ck_bn, relu=False)
    out_user = _tc_layer(h_user, sums2[N:], cnt_user,
                         c2cb_Ws, c2cb_bs, c2cb_Wn, c2cb_bn, relu=False)
    return (out_user, out_item)


# speedup vs baseline: 1.2068x; 1.0092x over previous
import functools

import jax
import jax.numpy as jnp
from jax import lax
from jax.experimental import pallas as pl
from jax.experimental.pallas import tpu as pltpu
from jax.experimental.pallas import tpu_sc as plsc

N = 10000
E = 320000
D = 128
NC = 2
NS = 16
EPT = E // NS
C = 40
NSLOT = 5
NGROUP = EPT // (C * NSLOT)
OWN = 624
NRB = OWN // C
RBL = OWN - NRB * C
REM = N - NS * OWN
CW = 16

_f32 = jnp.float32
_i32 = jnp.int32

def _sc_agg_body(with_gather, W, tabs, srci, dsti, zrow,
                 sums, acc, rows, sidx, didx, isem, gsem, ssem):
    cid = lax.axis_index("c")
    sid = lax.axis_index("s")
    row0 = sid * OWN
    obase = cid * N
    ebase = cid * E + sid * EPT

    pltpu.sync_copy(zrow, rows[0])
    if not with_gather:
        pltpu.sync_copy(srci, rows[1])
    for k in range(NRB):
        pltpu.sync_copy(rows[0], acc.at[pl.ds(row0 + k * C, C), :])
    rlo = row0 + NRB * C
    pltpu.sync_copy(rows[0].at[pl.ds(0, RBL), :], acc.at[pl.ds(rlo, RBL), :])

    @pl.when(sid == 0)
    def _():
        pltpu.sync_copy(rows[0].at[pl.ds(0, REM), :],
                        acc.at[pl.ds(NS * OWN, REM), :])

    plsc.subcore_barrier()

    def issue_idx(q, jj):
        off = ebase + jj * C
        if with_gather:
            pltpu.async_copy(srci.at[pl.ds(off, C)], sidx[q], isem[q])
        pltpu.async_copy(dsti.at[pl.ds(off, C)], didx[q], isem[q])

    def wait_idx(q):
        if with_gather:
            pltpu.make_async_copy(srci.at[pl.ds(0, C)], sidx[q],
                                  isem[q]).wait()
        pltpu.make_async_copy(dsti.at[pl.ds(0, C)], didx[q], isem[q]).wait()

    def issue_gather(s, q):
        if with_gather:
            pltpu.async_copy(tabs.at[sidx[q]], rows[s], gsem[s])

    def wait_gather(s, q):
        if with_gather:
            pltpu.make_async_copy(tabs.at[sidx[q]], rows[s], gsem[s]).wait()

    def srcbuf(s):
        return rows[s] if with_gather else rows[1]

    def issue_scatter(s, q):
        pltpu.async_copy(srcbuf(s), acc.at[didx[q]], ssem[s], add=True)

    def wait_scatter(s, q):
        pltpu.make_async_copy(srcbuf(s), acc.at[didx[q]], ssem[s]).wait()

    for s in range(NSLOT):
        issue_idx(s, s)

    def one_group(g, parity):
        qs = [parity * NSLOT + s for s in range(NSLOT)]
        qo = [(1 - parity) * NSLOT + s for s in range(NSLOT)]
        for s in range(NSLOT):
            @pl.when(g > 0)
            def _(s=s):
                wait_scatter(s, qo[s])
            wait_idx(qs[s])
            issue_gather(s, qs[s])
        for s in range(NSLOT):
            wait_gather(s, qs[s])
            issue_scatter(s, qs[s])
            @pl.when(g + 1 < NGROUP)
            def _(s=s, g=g):
                issue_idx(qo[s], (g + 1) * NSLOT + s)

    def loop_body(t, carry):
        one_group(2 * t, 0)
        one_group(2 * t + 1, 1)
        return carry

    lax.fori_loop(0, NGROUP // 2, loop_body, 0)
    if NGROUP % 2:
        one_group(NGROUP - 1, 0)
    lastp = (NGROUP - 1) % 2
    for s in range(NSLOT):
        wait_scatter(s, lastp * NSLOT + s)

    plsc.subcore_barrier()

    for k in range(NRB):
        r = row0 + k * C
        pltpu.sync_copy(acc.at[pl.ds(r, C), :], rows[0])
        pltpu.sync_copy(rows[0], sums.at[pl.ds(obase + r, C), :])
    pltpu.sync_copy(acc.at[pl.ds(rlo, RBL), :], rows[0].at[pl.ds(0, RBL), :])
    pltpu.sync_copy(rows[0].at[pl.ds(0, RBL), :],
                    sums.at[pl.ds(obase + rlo, RBL), :])

    @pl.when(sid == 0)
    def _():
        r = NS * OWN
        pltpu.sync_copy(acc.at[pl.ds(r, REM), :], rows[0].at[pl.ds(0, REM), :])
        pltpu.sync_copy(rows[0].at[pl.ds(0, REM), :],
                        sums.at[pl.ds(obase + r, REM), :])


def _make_sc_kernel(with_gather, W):
    out_type = jax.ShapeDtypeStruct((NC * N, W), _f32)
    scratch = [pltpu.VMEM_SHARED((N, W), _f32)]
    scratch += [pltpu.VMEM((C, W), _f32) for _ in range(NSLOT)]
    scratch += [pltpu.VMEM((C,), _i32) for _ in range(2 * NSLOT)]
    scratch += [pltpu.VMEM((C,), _i32) for _ in range(2 * NSLOT)]
    scratch += [pltpu.SemaphoreType.DMA for _ in range(2 * NSLOT)]
    scratch += [pltpu.SemaphoreType.DMA for _ in range(NSLOT)]
    scratch += [pltpu.SemaphoreType.DMA for _ in range(NSLOT)]

    def body(*refs):
        tabs, srci, dsti, zrow, sums = refs[:5]
        k = 5
        acc = refs[k]; k += 1
        rows = list(refs[k:k + NSLOT]); k += NSLOT
        sidx = list(refs[k:k + 2 * NSLOT]); k += 2 * NSLOT
        didx = list(refs[k:k + 2 * NSLOT]); k += 2 * NSLOT
        isem = list(refs[k:k + 2 * NSLOT]); k += 2 * NSLOT
        gsem = list(refs[k:k + NSLOT]); k += NSLOT
        ssem = list(refs[k:k + NSLOT]); k += NSLOT
        _sc_agg_body(with_gather, W, tabs, srci, dsti, zrow,
                     sums, acc, rows, sidx, didx, isem, gsem, ssem)

    mesh = plsc.VectorSubcoreMesh(core_axis_name="c", subcore_axis_name="s",
                                  num_cores=NC, num_subcores=NS)
    return pl.kernel(body, out_type=out_type, mesh=mesh, scratch_types=scratch)


_sc_agg = _make_sc_kernel(True, D)
_sc_count = _make_sc_kernel(False, D)


def _tc_body(x_ref, s_ref, c_ref, wsa_ref, bsa_ref, wna_ref, bna_ref,
             wsb_ref, bsb_ref, wnb_ref, bnb_ref, o_ref, *, relu, nblk):
    first = pl.program_id(0) < nblk
    ws = jnp.where(first, wsa_ref[...], wsb_ref[...])
    wn = jnp.where(first, wna_ref[...], wnb_ref[...])
    b = jnp.where(first, bsa_ref[...] + bna_ref[...],
                  bsb_ref[...] + bnb_ref[...])
    cnt = c_ref[:, 0:1]
    inv = 1.0 / jnp.maximum(cnt, 1.0)
    mean = s_ref[...] * inv
    acc = (jnp.dot(x_ref[...], ws, preferred_element_type=_f32)
           + jnp.dot(mean, wn, preferred_element_type=_f32) + b)
    if relu:
        acc = jnp.maximum(acc, 0.0)
    o_ref[...] = acc


def _tc_layer(x, s, cnt, wsa, bsa, wna, bna, wsb, bsb, wnb, bnb, relu):
    R = 1000
    W = wsa.shape[1]
    grid = (2 * N // R,)
    wspec = pl.BlockSpec((D, W), lambda i: (0, 0))
    bspec = pl.BlockSpec((1, W), lambda i: (0, 0))
    return pl.pallas_call(
        functools.partial(_tc_body, relu=relu, nblk=N // R),
        grid=grid,
        in_specs=[
            pl.BlockSpec((R, D), lambda i: (i, 0)),
            pl.BlockSpec((R, D), lambda i: (i, 0)),
            pl.BlockSpec((R, CW), lambda i: (i, 0)),
            wspec, bspec, wspec, bspec,
            wspec, bspec, wspec, bspec,
        ],
        out_specs=pl.BlockSpec((R, W), lambda i: (i, 0)),
        out_shape=jax.ShapeDtypeStruct((2 * N, W), _f32),
    )(x, s, cnt, wsa, bsa.reshape(1, W), wna, bna.reshape(1, W),
      wsb, bsb.reshape(1, W), wnb, bnb.reshape(1, W))


def kernel(x_user, x_item, edge_index_clicks, edge_index_clicked_by,
           c1ck_Ws, c1ck_bs, c1ck_Wn, c1ck_bn,
           c1cb_Ws, c1cb_bs, c1cb_Wn, c1cb_bn,
           c2ck_Ws, c2ck_bs, c2ck_Wn, c2ck_bn,
           c2cb_Ws, c2cb_bs, c2cb_Wn, c2cb_bn):
    srci = jnp.concatenate(
        [edge_index_clicks[0], edge_index_clicked_by[0] + N]).astype(_i32)
    dsti = jnp.concatenate(
        [edge_index_clicks[1], edge_index_clicked_by[1]]).astype(_i32)
    zrow = jnp.zeros((C, D), _f32)
    orow = jnp.ones((C, D), _f32)

    csum = _sc_count(zrow, orow, dsti, zrow)
    cnts = csum[:, :CW]

    tabs1 = jnp.concatenate([x_user, x_item], axis=0)
    sums1 = _sc_agg(tabs1, srci, dsti, zrow)

    xs = jnp.concatenate([x_item, x_user], axis=0)
    h = _tc_layer(xs, sums1, cnts,
                  c1ck_Ws, c1ck_bs, c1ck_Wn, c1ck_bn,
                  c1cb_Ws, c1cb_bs, c1cb_Wn, c1cb_bn, relu=True)

    tabs2 = jnp.concatenate([h[N:], h[:N]], axis=0)
    sums2 = _sc_agg(tabs2, srci, dsti, zrow)

    out = _tc_layer(h, sums2, cnts,
                    c2ck_Ws, c2ck_bs, c2ck_Wn, c2ck_bn,
                    c2cb_Ws, c2cb_bs, c2cb_Wn, c2cb_bn, relu=False)
    return (out[N:], out[:N])
